# baseline (speedup 1.0000x reference)
import jax
import jax.numpy as jnp
from jax import lax
from jax.experimental import pallas as pl
from jax.experimental.pallas import tpu as pltpu
from jax.experimental.pallas import tpu_sc as plsc

N = 50000
K = 32
KP = K + 1
L = 16
NW = 32
GROUPS_PER_TILE = 98
PER_W = GROUPS_PER_TILE * L
NCHUNK = 7
GROUPS_PER_CHUNK = GROUPS_PER_TILE // NCHUNK
CHUNK_NODES = GROUPS_PER_CHUNK * L
CHUNK_E = CHUNK_NODES * K
CHUNK_P = CHUNK_NODES * KP

MEAN_PER_SUB = 196


def _moran_body(x_hbm, pk_hbm, out_hbm,
                x_v, pk_a, pk_b, pk_c, pk_p,
                out_v, red_v, shared_red,
                sem_x, sem_pk):
    cid = lax.axis_index("c")
    sid = lax.axis_index("s")
    wid = sid * 2 + cid
    base = jnp.where(wid == NW - 1, N - PER_W, wid * PER_W)
    ebase = base * K

    pk_bufs = (pk_a, pk_b, pk_c)

    def issue(ci):
        off = ebase + ci * CHUNK_E
        return pltpu.async_copy(pk_hbm.at[pl.ds(off, CHUNK_E)],
                                pk_bufs[ci % 3], sem_pk)

    cp_x = pltpu.async_copy(x_hbm, x_v, sem_x)
    pending = {0: issue(0), 1: issue(1), 2: issue(2)}

    iota = lax.iota(jnp.int32, L)
    iota_kp = iota * KP
    lo16 = jnp.full((L,), 0xFFFF, jnp.int32)
    hi16 = jnp.full((L,), -65536, jnp.int32)
    z = jnp.zeros((L,), jnp.float32)

    def compute_mean():
        mstart = sid * MEAN_PER_SUB * L
        def mean_body(i, accs):
            b = mstart + i * (4 * L)
            a0, a1, a2, a3 = accs
            a0 = a0 + x_v[pl.ds(b, L)]
            a1 = a1 + x_v[pl.ds(b + L, L)]
            a2 = a2 + x_v[pl.ds(b + 2 * L, L)]
            a3 = a3 + x_v[pl.ds(b + 3 * L, L)]
            return (a0, a1, a2, a3)
        nquad = jnp.where(sid == 15, 45, 49)
        accs = lax.fori_loop(0, nquad, mean_body, (z, z, z, z))
        part = accs[0] + accs[1] + accs[2] + accs[3]

        def mean_tail(i, p):
            return p + x_v[pl.ds(mstart + (180 + i) * L, L)]
        part = jnp.where(sid == 15, lax.fori_loop(0, 5, mean_tail, z), z) + part

        red_v[pl.ds(0, L)] = part
        pltpu.sync_copy(red_v.at[pl.ds(0, L)],
                        shared_red.at[pl.ds(sid * L, L)])
        plsc.subcore_barrier()
        pltpu.sync_copy(shared_red, red_v)
        tot = z
        for r in range(L):
            tot = tot + red_v[pl.ds(r * L, L)]
        s = tot[0]
        for i in range(1, L):
            s = s + tot[i]
        return s * (1.0 / N)

    m = None

    def make_compute_group(m):
        def compute_group(idx_base, goff):
            z16 = z
            acc = [[z16, z16, z16], [z16, z16, z16]]
            for j in range(K):
                pv = plsc.load_gather(pk_p, [idx_base + j])
                nid = pv & lo16
                w = plsc.bitcast(pv & hi16, jnp.float32)
                xg = plsc.load_gather(x_v, [nid])
                t = w * xg
                a = acc[j % 2]
                a[0] = a[0] + w
                a[1] = a[1] + t
                a[2] = a[2] + t * xg
            sw = acc[0][0] + acc[1][0]
            swx = acc[0][1] + acc[1][1]
            swxx = acc[0][2] + acc[1][2]
            own = x_v[pl.ds(base + goff, L)]
            xa = own - m
            num = swx - m * sw
            den = swxx - m * (2.0 * swx - m * sw)
            out_v[pl.ds(goff, L)] = xa * num * (K - 1.0) / den
        return compute_group

    for ci in range(NCHUNK):
        pb = pk_bufs[ci % 3]
        pending.pop(ci).wait()

        def repack(q, _, pb=pb):
            s0 = q * (4 * K)
            d0 = q * (4 * KP)
            for u in range(4):
                se = s0 + u * K
                de = d0 + u * KP
                pk_p[pl.ds(de, L)] = pb[pl.ds(se, L)]
                pk_p[pl.ds(de + L, L)] = pb[pl.ds(se + L, L)]
            return 0
        lax.fori_loop(0, CHUNK_NODES // 4, repack, 0)

        if ci == 0:
            cp_x.wait()
            m = compute_mean()
            compute_group = make_compute_group(m)

        def grp(g, _, ci=ci):
            goff2 = (ci * GROUPS_PER_CHUNK + g * 2) * L
            compute_group((g * 2) * (L * KP) + iota_kp, goff2)
            compute_group((g * 2 + 1) * (L * KP) + iota_kp, goff2 + L)
            return 0

        lax.fori_loop(0, GROUPS_PER_CHUNK // 2, grp, 0)
        if ci + 3 < NCHUNK:
            pending[ci + 3] = issue(ci + 3)

    pltpu.sync_copy(out_v, out_hbm.at[pl.ds(base, PER_W)])


@jax.jit
def _moran_sc(x, pk_flat):
    mesh = plsc.VectorSubcoreMesh(core_axis_name="c", subcore_axis_name="s")
    return pl.kernel(
        _moran_body,
        out_type=jax.ShapeDtypeStruct((N,), jnp.float32),
        mesh=mesh,
        compiler_params=pltpu.CompilerParams(needs_layout_passes=False),
        scratch_types=[
            pltpu.VMEM((N,), jnp.float32),
            pltpu.VMEM((CHUNK_E,), jnp.int32),
            pltpu.VMEM((CHUNK_E,), jnp.int32),
            pltpu.VMEM((CHUNK_E,), jnp.int32),
            pltpu.VMEM((CHUNK_P,), jnp.int32),
            pltpu.VMEM((PER_W,), jnp.float32),
            pltpu.VMEM((16 * L,), jnp.float32),
            pltpu.VMEM_SHARED((16 * L,), jnp.float32),
            pltpu.SemaphoreType.DMA,
            pltpu.SemaphoreType.DMA,
        ],
    )(x, pk_flat)


def kernel(X, neighbor_weights, neighbor_ids):
    wu = jax.lax.bitcast_convert_type(neighbor_weights, jnp.uint32)
    wbf = (wu + 0x7FFF + ((wu >> 16) & 1)) & jnp.uint32(0xFFFF0000)
    pk = wbf | neighbor_ids.astype(jnp.uint32)
    pk = jax.lax.optimization_barrier(pk)
    pk_flat = jax.lax.bitcast_convert_type(pk, jnp.int32).reshape(-1)
    return _moran_sc(X, pk_flat)


# speedup vs baseline: 1.0057x; 1.0057x over previous
import jax
import jax.numpy as jnp
from jax import lax
from jax.experimental import pallas as pl
from jax.experimental.pallas import tpu as pltpu
from jax.experimental.pallas import tpu_sc as plsc

N = 50000
K = 32
KP = K + 1
L = 16
NW = 32
GROUPS_PER_TILE = 98
PER_W = GROUPS_PER_TILE * L
NCHUNK = 7
GROUPS_PER_CHUNK = GROUPS_PER_TILE // NCHUNK
CHUNK_NODES = GROUPS_PER_CHUNK * L
CHUNK_E = CHUNK_NODES * K
CHUNK_P = CHUNK_NODES * KP

MEAN_PER_SUB = 196


def _moran_body(x_hbm, pk_hbm, out_hbm,
                x_v, pk_a, pk_b, pk_c, pk_p,
                out_v, red_v, shared_red,
                sem_x, sem_pk):
    cid = lax.axis_index("c")
    sid = lax.axis_index("s")
    wid = sid * 2 + cid
    base = jnp.where(wid == NW - 1, N - PER_W, wid * PER_W)
    ebase = base * K

    pk_bufs = (pk_a, pk_b, pk_c)

    def issue(ci):
        off = ebase + ci * CHUNK_E
        return pltpu.async_copy(pk_hbm.at[pl.ds(off, CHUNK_E)],
                                pk_bufs[ci % 3], sem_pk)

    cp_x = pltpu.async_copy(x_hbm, x_v, sem_x)
    pending = {0: issue(0), 1: issue(1), 2: issue(2)}

    iota = lax.iota(jnp.int32, L)
    iota_kp = iota * KP
    lo16 = jnp.full((L,), 0xFFFF, jnp.int32)
    hi16 = jnp.full((L,), -65536, jnp.int32)
    z = jnp.zeros((L,), jnp.float32)

    def compute_mean():
        mstart = sid * MEAN_PER_SUB * L
        def mean_body(i, accs):
            b = mstart + i * (4 * L)
            a0, a1, a2, a3 = accs
            a0 = a0 + x_v[pl.ds(b, L)]
            a1 = a1 + x_v[pl.ds(b + L, L)]
            a2 = a2 + x_v[pl.ds(b + 2 * L, L)]
            a3 = a3 + x_v[pl.ds(b + 3 * L, L)]
            return (a0, a1, a2, a3)
        nquad = jnp.where(sid == 15, 45, 49)
        accs = lax.fori_loop(0, nquad, mean_body, (z, z, z, z))
        part = accs[0] + accs[1] + accs[2] + accs[3]

        def mean_tail(i, p):
            return p + x_v[pl.ds(mstart + (180 + i) * L, L)]
        part = jnp.where(sid == 15, lax.fori_loop(0, 5, mean_tail, z), z) + part

        red_v[pl.ds(0, L)] = part
        pltpu.sync_copy(red_v.at[pl.ds(0, L)],
                        shared_red.at[pl.ds(sid * L, L)])
        plsc.subcore_barrier()
        pltpu.sync_copy(shared_red, red_v)
        tot = z
        for r in range(L):
            tot = tot + red_v[pl.ds(r * L, L)]
        s = tot[0]
        for i in range(1, L):
            s = s + tot[i]
        return s * (1.0 / N)

    m = None

    def make_compute_group(m):
        def compute_group(idx_base, goff):
            z16 = z
            acc = [[z16, z16, z16], [z16, z16, z16]]
            for j in range(K):
                pv = plsc.load_gather(pk_p, [idx_base + j])
                nid = pv & lo16
                w = plsc.bitcast(pv & hi16, jnp.float32)
                xg = plsc.load_gather(x_v, [nid])
                t = w * xg
                a = acc[j % 2]
                a[0] = a[0] + w
                a[1] = a[1] + t
                a[2] = a[2] + t * xg
            sw = acc[0][0] + acc[1][0]
            swx = acc[0][1] + acc[1][1]
            swxx = acc[0][2] + acc[1][2]
            own = x_v[pl.ds(base + goff, L)]
            xa = own - m
            num = swx - m * sw
            den = swxx - m * (2.0 * swx - m * sw)
            out_v[pl.ds(goff, L)] = xa * num * (K - 1.0) / den
        return compute_group

    for ci in range(NCHUNK):
        pb = pk_bufs[ci % 3]
        pending.pop(ci).wait()

        def repack(q, _, pb=pb):
            s0 = q * (4 * K)
            d0 = q * (4 * KP)
            for u in range(4):
                se = s0 + u * K
                de = d0 + u * KP
                pk_p[pl.ds(de, L)] = pb[pl.ds(se, L)]
                pk_p[pl.ds(de + L, L)] = pb[pl.ds(se + L, L)]
            return 0
        lax.fori_loop(0, CHUNK_NODES // 4, repack, 0)

        if ci == 0:
            cp_x.wait()
            m = compute_mean()
            compute_group = make_compute_group(m)

        def grp(g, _, ci=ci):
            goff2 = (ci * GROUPS_PER_CHUNK + g * 2) * L
            compute_group((g * 2) * (L * KP) + iota_kp, goff2)
            compute_group((g * 2 + 1) * (L * KP) + iota_kp, goff2 + L)
            return 0

        lax.fori_loop(0, GROUPS_PER_CHUNK // 2, grp, 0)
        if ci + 3 < NCHUNK:
            pending[ci + 3] = issue(ci + 3)

    pltpu.sync_copy(out_v, out_hbm.at[pl.ds(base, PER_W)])


@jax.jit
def _moran_sc(x, pk_flat):
    mesh = plsc.VectorSubcoreMesh(core_axis_name="c", subcore_axis_name="s")
    return pl.kernel(
        _moran_body,
        out_type=jax.ShapeDtypeStruct((N,), jnp.float32),
        mesh=mesh,
        compiler_params=pltpu.CompilerParams(needs_layout_passes=False),
        scratch_types=[
            pltpu.VMEM((N,), jnp.float32),
            pltpu.VMEM((CHUNK_E,), jnp.int32),
            pltpu.VMEM((CHUNK_E,), jnp.int32),
            pltpu.VMEM((CHUNK_E,), jnp.int32),
            pltpu.VMEM((CHUNK_P,), jnp.int32),
            pltpu.VMEM((PER_W,), jnp.float32),
            pltpu.VMEM((16 * L,), jnp.float32),
            pltpu.VMEM_SHARED((16 * L,), jnp.float32),
            pltpu.SemaphoreType.DMA,
            pltpu.SemaphoreType.DMA,
        ],
    )(x, pk_flat)


def kernel(X, neighbor_weights, neighbor_ids):
    wu = jax.lax.bitcast_convert_type(neighbor_weights, jnp.uint32)
    wbf = (wu + 0x7FFF + ((wu >> 16) & 1)) & jnp.uint32(0xFFFF0000)
    pk = wbf | neighbor_ids.astype(jnp.uint32)
    pk_flat = jax.lax.bitcast_convert_type(pk, jnp.int32).reshape(-1)
    return _moran_sc(X, pk_flat)
